# baseline (speedup 1.0000x reference)
import jax
import jax.numpy as jnp
from jax import lax
from jax.experimental import pallas as pl
from jax.experimental.pallas import tpu as pltpu
from jax.experimental.pallas import tpu_sc as plsc

NC = 2
NS = 16
L = 16
NW = NC * NS

B = 16384
D = 64
BPW = B // NW
CH = 128
NCH = BPW // CH


def _decoder_body(z2_hbm, h_hbm, t_hbm, out_hbm,
                  idx_v, gat_v, par_v, hrows, trows, out_v, sem):
    wid = lax.axis_index("s") * NC + lax.axis_index("c")
    base = wid * BPW

    pltpu.sync_copy(h_hbm.at[pl.ds(base, BPW)], idx_v.at[pl.ds(0, BPW)])
    pltpu.sync_copy(t_hbm.at[pl.ds(base, BPW)], idx_v.at[pl.ds(BPW, BPW)])

    for k in range(2 * BPW // L):
        v = idx_v[pl.ds(k * L, L)]
        gat_v[k * L // CH, pl.ds((k * L) % CH, L)] = v >> 1
        par_v[pl.ds(k * L, L)] = v & 1

    def fire(k, buf):
        return (
            pltpu.async_copy(z2_hbm.at[gat_v.at[k]], hrows.at[buf], sem),
            pltpu.async_copy(z2_hbm.at[gat_v.at[NCH + k]], trows.at[buf], sem),
        )

    lanes = lax.iota(jnp.int32, L)

    def compute(k, buf):
        def group(g, carry):
            ph = par_v[pl.ds(k * CH + g * L, L)] > 0
            pt = par_v[pl.ds(BPW + k * CH + g * L, L)] > 0
            rll = jnp.zeros((L,), jnp.float32)
            rlr = jnp.zeros((L,), jnp.float32)
            rrl = jnp.zeros((L,), jnp.float32)
            rrr = jnp.zeros((L,), jnp.float32)
            for j in range(L):
                r = g * L + j
                sll = jnp.zeros((L,), jnp.float32)
                slr = jnp.zeros((L,), jnp.float32)
                srl = jnp.zeros((L,), jnp.float32)
                srr = jnp.zeros((L,), jnp.float32)
                for c in range(D // L):
                    hlo = hrows[buf, r, pl.ds(c * L, L)]
                    hhi = hrows[buf, r, pl.ds(D + c * L, L)]
                    tlo = trows[buf, r, pl.ds(c * L, L)]
                    thi = trows[buf, r, pl.ds(D + c * L, L)]
                    sll = sll + hlo * tlo
                    slr = slr + hlo * thi
                    srl = srl + hhi * tlo
                    srr = srr + hhi * thi
                m = lanes == j
                rll = jnp.where(m, jnp.sum(sll), rll)
                rlr = jnp.where(m, jnp.sum(slr), rlr)
                rrl = jnp.where(m, jnp.sum(srl), rrl)
                rrr = jnp.where(m, jnp.sum(srr), rrr)
            res = jnp.where(ph, jnp.where(pt, rrr, rrl),
                            jnp.where(pt, rlr, rll))
            out_v[pl.ds(k * CH + g * L, L)] = res
            return carry

        lax.fori_loop(0, CH // L, group, 0)

    descs = {0: fire(0, 0)}
    for k in range(NCH):
        if k + 1 < NCH:
            descs[k + 1] = fire(k + 1, (k + 1) % 2)
        for d in descs.pop(k):
            d.wait()
        compute(k, k % 2)

    pltpu.sync_copy(out_v, out_hbm.at[pl.ds(base, BPW)])


def _decode(z2, h, t):
    mesh = plsc.VectorSubcoreMesh(core_axis_name="c", subcore_axis_name="s",
                                  num_cores=NC, num_subcores=NS)
    return pl.kernel(
        _decoder_body,
        out_type=jax.ShapeDtypeStruct((B,), jnp.float32),
        mesh=mesh,
        compiler_params=pltpu.CompilerParams(needs_layout_passes=False),
        scratch_types=[
            pltpu.VMEM((2 * BPW,), jnp.int32),
            pltpu.VMEM((2 * NCH, CH), jnp.int32),
            pltpu.VMEM((2 * BPW,), jnp.int32),
            pltpu.VMEM((2, CH, 2 * D), jnp.float32),
            pltpu.VMEM((2, CH, 2 * D), jnp.float32),
            pltpu.VMEM((BPW,), jnp.float32),
            pltpu.SemaphoreType.DMA,
        ],
    )(z2, h, t)


def kernel(z, triples):
    h = triples[:, 0].astype(jnp.int32)
    t = triples[:, 2].astype(jnp.int32)
    return _decode(z.reshape(500000, 128), h, t)
